# baseline (speedup 1.0000x reference)
import functools

import jax
import jax.numpy as jnp
from jax import lax
from jax.experimental import pallas as pl
from jax.experimental.pallas import tpu as pltpu
from jax.experimental.pallas import tpu_sc as plsc

B, T, L, V = 1024, 200, 32, 100000
BT = B * T
N_STATIC, N_KNOWN_CAT, N_KNOWN_REAL, N_OBS = 4, 2, 4, 3
KNOWN_F = N_KNOWN_REAL + N_KNOWN_CAT
KW = L * KNOWN_F
OW = L * N_OBS

NC, NS = 2, 16
NW = NC * NS

G_ROWS = N_KNOWN_CAT * BT
S_ROWS = N_STATIC * B
ROWS_PER_W = G_ROWS // NW
CHUNK = 1280
FPC = CHUNK // 128
NCHUNK = ROWS_PER_W // CHUNK
IDXR_PER_W = ROWS_PER_W // 128
S_PER_W = S_ROWS // NW

_mesh = plsc.VectorSubcoreMesh(core_axis_name="c", subcore_axis_name="s")


@functools.partial(
    pl.kernel,
    mesh=_mesh,
    out_type=[
        jax.ShapeDtypeStruct((G_ROWS, L), jnp.float32),
        jax.ShapeDtypeStruct((S_ROWS, L), jnp.float32),
    ],
    scratch_types=[
        pltpu.VMEM((IDXR_PER_W, 128), jnp.int32),
        pltpu.VMEM((CHUNK, L), jnp.float32),
        pltpu.VMEM((1, 128), jnp.int32),
        pltpu.VMEM((S_PER_W, L), jnp.float32),
        pltpu.SemaphoreType.DMA,
    ],
    compiler_params=pltpu.CompilerParams(use_tc_tiling_on_sc=False),
)
def _sc_gather(ktab, kidx, stab, sidx, g_out, s_out, idx_v, rows_v, sidx_v, srows_v, sem):
    wid = lax.axis_index("s") * NC + lax.axis_index("c")
    pltpu.sync_copy(sidx.at[wid], sidx_v)
    pltpu.async_copy(stab.at[sidx_v.at[0]], srows_v, sem).wait()
    pltpu.sync_copy(srows_v, s_out.at[pl.ds(wid * S_PER_W, S_PER_W)])

    row_base = wid * ROWS_PER_W
    pltpu.sync_copy(kidx.at[wid], idx_v)

    def chunk_body(c, carry):
        copies = [
            pltpu.async_copy(
                ktab.at[idx_v.at[c * FPC + j]], rows_v.at[pl.ds(j * 128, 128)], sem
            )
            for j in range(FPC)
        ]
        for cp in copies:
            cp.wait()
        pltpu.sync_copy(rows_v, g_out.at[pl.ds(row_base + c * CHUNK, CHUNK)])
        return carry

    lax.fori_loop(0, NCHUNK, chunk_body, 0)


def _known_body(kr_ref, g_ref, p_ref, c_ref, b_ref, o_ref):
    o_ref[...] = (
        lax.dot(kr_ref[...], p_ref[...], precision=lax.Precision.HIGHEST,
                preferred_element_type=jnp.float32)
        + lax.dot(g_ref[...], c_ref[...], precision=lax.Precision.HIGHEST,
                  preferred_element_type=jnp.float32)
        + b_ref[...]
    )


def _obs_body(x_ref, po_ref, bo_ref, o_ref):
    o_ref[...] = (
        lax.dot(x_ref[...], po_ref[...], precision=lax.Precision.HIGHEST,
                preferred_element_type=jnp.float32)
        + bo_ref[...]
    )


BTB = 2048


def kernel(static, known_real, known_categorical, observed, static_tables,
           known_tables, known_dense_w, known_dense_b, observed_dense_w,
           observed_dense_b):
    f32, i32 = jnp.float32, jnp.int32

    kidx = known_categorical.astype(i32) + jnp.arange(N_KNOWN_CAT, dtype=i32) * V
    kidx = kidx.reshape(NW, IDXR_PER_W, 128)
    sidx = static[:, 0, :].astype(i32) + jnp.arange(N_STATIC, dtype=i32) * V
    sidx = sidx.reshape(NW, 1, 128)

    ktab = known_tables.reshape(N_KNOWN_CAT * V, L)
    stab = static_tables.reshape(N_STATIC * V, L)

    g2, s2 = _sc_gather(ktab, kidx, stab, sidx)
    static_embeds = s2.reshape(B, N_STATIC, L)
    g2 = g2.reshape(BT, N_KNOWN_CAT * L)

    lr = jnp.arange(L)
    fr = jnp.arange(N_KNOWN_REAL)
    cols_r = lr[None, :] * KNOWN_F + fr[:, None]
    P = jnp.zeros((N_KNOWN_REAL, KW), f32).at[
        fr[:, None], cols_r].set(known_dense_w.reshape(N_KNOWN_REAL, L))
    brow = jnp.zeros((1, KW), f32).at[0, cols_r.reshape(-1)].set(
        known_dense_b.reshape(-1))
    tr = jnp.arange(N_KNOWN_CAT)
    rows_c = (tr[:, None] * L + lr[None, :]).reshape(-1)
    cols_c = (lr[None, :] * KNOWN_F + N_KNOWN_REAL + tr[:, None]).reshape(-1)
    C = jnp.zeros((N_KNOWN_CAT * L, KW), f32).at[rows_c, cols_c].set(1.0)

    fo = jnp.arange(N_OBS)
    cols_o = lr[None, :] * N_OBS + fo[:, None]
    Po = jnp.zeros((N_OBS, OW), f32).at[
        fo[:, None], cols_o].set(observed_dense_w.reshape(N_OBS, L))
    brow_o = jnp.zeros((1, OW), f32).at[0, cols_o.reshape(-1)].set(
        observed_dense_b.reshape(-1))

    obs2d = pl.pallas_call(
        _obs_body,
        grid=(BT // BTB,),
        in_specs=[
            pl.BlockSpec((BTB, N_OBS), lambda i: (i, 0)),
            pl.BlockSpec((N_OBS, OW), lambda i: (0, 0)),
            pl.BlockSpec((1, OW), lambda i: (0, 0)),
        ],
        out_specs=pl.BlockSpec((BTB, OW), lambda i: (i, 0)),
        out_shape=jax.ShapeDtypeStruct((BT, OW), f32),
    )(observed.reshape(BT, N_OBS), Po, brow_o)

    out2d = pl.pallas_call(
        _known_body,
        grid=(BT // BTB,),
        in_specs=[
            pl.BlockSpec((BTB, N_KNOWN_REAL), lambda i: (i, 0)),
            pl.BlockSpec((BTB, N_KNOWN_CAT * L), lambda i: (i, 0)),
            pl.BlockSpec((N_KNOWN_REAL, KW), lambda i: (0, 0)),
            pl.BlockSpec((N_KNOWN_CAT * L, KW), lambda i: (0, 0)),
            pl.BlockSpec((1, KW), lambda i: (0, 0)),
        ],
        out_specs=pl.BlockSpec((BTB, KW), lambda i: (i, 0)),
        out_shape=jax.ShapeDtypeStruct((BT, KW), f32),
    )(known_real.reshape(BT, N_KNOWN_REAL), g2, P, C, brow)

    known = out2d.reshape(B, T, L, KNOWN_F)
    observed_embeds = obs2d.reshape(B, T, L, N_OBS)
    return (static_embeds, known, observed_embeds)


# speedup vs baseline: 2.9042x; 2.9042x over previous
import functools

import jax
import jax.numpy as jnp
from jax import lax
from jax.experimental import pallas as pl
from jax.experimental.pallas import tpu as pltpu
from jax.experimental.pallas import tpu_sc as plsc

B, T, L, V = 1024, 200, 32, 100000
BT = B * T
N_STATIC, N_KNOWN_CAT, N_KNOWN_REAL, N_OBS = 4, 2, 4, 3
KNOWN_F = N_KNOWN_REAL + N_KNOWN_CAT

NC, NS = 2, 16
NW = NC * NS

HU = T * N_KNOWN_CAT * 2
HU_PER_W = HU // NW

G_ROWS = T * N_KNOWN_CAT * (L // 8) * (B // 128) * 8
S_ROWS2 = N_STATIC * (L // 8) * (B // 128) * 8
KNOWN_ROWS = T * KNOWN_F * (L // 8) * (B // 128) * 8
OBS_ROWS = T * N_OBS * (L // 8) * (B // 128) * 8

_mesh = plsc.VectorSubcoreMesh(core_axis_name="c", subcore_axis_name="s")


def _transpose_unit(tab, idx_rows, rows_v, tb, sem, n_rows):
    fires = n_rows // 128
    copies = [
        pltpu.async_copy(
            tab.at[idx_rows.at[j]], rows_v.at[pl.ds(j * 128, 128)], sem
        )
        for j in range(fires)
    ]
    for cp in copies:
        cp.wait()
    lane = lax.iota(jnp.int32, 16)
    i0a = lane // 8
    i1a = lane % 8
    zero = jnp.zeros((16,), jnp.int32)

    def bgl_body(bgl, _):
        def blo_body(blo, __):
            r = bgl * 128 + blo
            v0 = rows_v[r, pl.ds(0, 16)]
            v1 = rows_v[r, pl.ds(16, 16)]
            d1 = i1a + bgl * 8
            d2 = zero + blo
            plsc.store_scatter(tb, [i0a, d1, d2], v0)
            plsc.store_scatter(tb, [i0a + 2, d1, d2], v1)
            return __

        return lax.fori_loop(0, 128, blo_body, _, unroll=2)

    lax.fori_loop(0, fires, bgl_body, 0)


@functools.partial(
    pl.kernel,
    mesh=_mesh,
    out_type=[
        jax.ShapeDtypeStruct((G_ROWS, 128), jnp.float32),
        jax.ShapeDtypeStruct((S_ROWS2, 128), jnp.float32),
    ],
    scratch_types=[
        pltpu.VMEM((4, 128), jnp.int32),
        pltpu.VMEM((512, 32), jnp.float32),
        pltpu.VMEM((4, 32, 128), jnp.float32),
        pltpu.SemaphoreType.DMA,
    ],
    compiler_params=pltpu.CompilerParams(
        use_tc_tiling_on_sc=False, needs_layout_passes=False),
)
def _sc_gather(ktab, kidx, stab, sidx, g_out, s_out, idx_v, rows_v, tb, sem):
    wid = lax.axis_index("s") * NC + lax.axis_index("c")

    def unit_body(k, _):
        u = wid * HU_PER_W + k
        t2 = u // 2
        bh = u % 2
        pltpu.sync_copy(kidx.at[u], idx_v)
        _transpose_unit(ktab, idx_v, rows_v, tb, sem, 512)
        base = t2 * 256 + bh * 32
        for lg in range(4):
            pltpu.sync_copy(tb.at[lg], g_out.at[pl.ds(base + lg * 64, 32)])
        return _

    lax.fori_loop(0, HU_PER_W, unit_body, 0)

    @pl.when(wid < 8)
    def _():
        f = wid // 2
        bh = wid % 2
        pltpu.sync_copy(sidx.at[wid], idx_v)
        _transpose_unit(stab, idx_v, rows_v, tb, sem, 512)
        base = f * 256 + bh * 32
        for lg in range(4):
            pltpu.sync_copy(tb.at[lg], s_out.at[pl.ds(base + lg * 64, 32)])


def _known_body(kr_ref, g_ref, w_ref, b_ref, o_ref):
    kr = kr_ref[...]
    kr_exp = jnp.broadcast_to(
        kr.reshape(4, 1, 8, 1, 128), (4, 4, 8, 8, 128)
    ).reshape(1024, 128)
    o_ref[pl.ds(0, 1024), :] = w_ref[...] * kr_exp + b_ref[...]
    o_ref[pl.ds(1024, 512), :] = g_ref[...]


def _obs_body(x_ref, w_ref, b_ref, o_ref):
    x = x_ref[...]
    x_exp = jnp.broadcast_to(
        x.reshape(3, 1, 8, 1, 128), (3, 4, 8, 8, 128)
    ).reshape(768, 128)
    o_ref[...] = w_ref[...] * x_exp + b_ref[...]


def kernel(static, known_real, known_categorical, observed, static_tables,
           known_tables, known_dense_w, known_dense_b, observed_dense_w,
           observed_dense_b):
    f32, i32 = jnp.float32, jnp.int32

    kidx = known_categorical.astype(i32) + jnp.arange(N_KNOWN_CAT, dtype=i32) * V
    kidx = kidx.transpose(1, 2, 0).reshape(HU, 4, 128)
    sidx = static[:, 0, :].astype(i32) + jnp.arange(N_STATIC, dtype=i32) * V
    sidx = sidx.T.reshape(2 * N_STATIC, 4, 128)

    ktab = known_tables.reshape(N_KNOWN_CAT * V, L)
    stab = static_tables.reshape(N_STATIC * V, L)

    krF = known_real.transpose(1, 2, 0).reshape(T * N_KNOWN_REAL * 8, 128)
    obsF = observed.transpose(1, 2, 0).reshape(T * N_OBS * 8, 128)

    w = known_dense_w.reshape(N_KNOWN_REAL, L)
    bw = known_dense_b.reshape(N_KNOWN_REAL, L)
    w_big = jnp.broadcast_to(
        w.reshape(4, 4, 1, 8, 1), (4, 4, 8, 8, 128)).reshape(1024, 128)
    b_big = jnp.broadcast_to(
        bw.reshape(4, 4, 1, 8, 1), (4, 4, 8, 8, 128)).reshape(1024, 128)
    wo = observed_dense_w.reshape(N_OBS, L)
    bo = observed_dense_b.reshape(N_OBS, L)
    wo_big = jnp.broadcast_to(
        wo.reshape(3, 4, 1, 8, 1), (3, 4, 8, 8, 128)).reshape(768, 128)
    bo_big = jnp.broadcast_to(
        bo.reshape(3, 4, 1, 8, 1), (3, 4, 8, 8, 128)).reshape(768, 128)

    g2, s2 = _sc_gather(ktab, kidx, stab, sidx)

    out_o = pl.pallas_call(
        _obs_body,
        grid=(T,),
        in_specs=[
            pl.BlockSpec((N_OBS * 8, 128), lambda i: (i, 0)),
            pl.BlockSpec((768, 128), lambda i: (0, 0)),
            pl.BlockSpec((768, 128), lambda i: (0, 0)),
        ],
        out_specs=pl.BlockSpec((768, 128), lambda i: (i, 0)),
        out_shape=jax.ShapeDtypeStruct((OBS_ROWS, 128), f32),
    )(obsF, wo_big, bo_big)

    out2 = pl.pallas_call(
        _known_body,
        grid=(T,),
        in_specs=[
            pl.BlockSpec((N_KNOWN_REAL * 8, 128), lambda i: (i, 0)),
            pl.BlockSpec((512, 128), lambda i: (i, 0)),
            pl.BlockSpec((1024, 128), lambda i: (0, 0)),
            pl.BlockSpec((1024, 128), lambda i: (0, 0)),
        ],
        out_specs=pl.BlockSpec((1536, 128), lambda i: (i, 0)),
        out_shape=jax.ShapeDtypeStruct((KNOWN_ROWS, 128), f32),
    )(krF, g2, w_big, b_big)

    known = (out2.reshape(T, KNOWN_F, 4, 8, 8, 128)
             .transpose(3, 5, 0, 2, 4, 1).reshape(B, T, L, KNOWN_F))
    observed_embeds = (out_o.reshape(T, N_OBS, 4, 8, 8, 128)
                       .transpose(3, 5, 0, 2, 4, 1).reshape(B, T, L, N_OBS))
    static_embeds = (s2.reshape(N_STATIC, 4, 8, 8, 128)
                     .transpose(2, 4, 0, 1, 3).reshape(B, N_STATIC, L))
    return (static_embeds, known, observed_embeds)


# speedup vs baseline: 3.0884x; 1.0634x over previous
import functools

import jax
import jax.numpy as jnp
from jax import lax
from jax.experimental import pallas as pl
from jax.experimental.pallas import tpu as pltpu
from jax.experimental.pallas import tpu_sc as plsc

B, T, L, V = 1024, 200, 32, 100000
BT = B * T
N_STATIC, N_KNOWN_CAT, N_KNOWN_REAL, N_OBS = 4, 2, 4, 3
KNOWN_F = N_KNOWN_REAL + N_KNOWN_CAT

NC, NS = 2, 16
NW = NC * NS

HU_PER_W = (T * N_KNOWN_CAT * 2) // NW

G_ROWS = T * N_KNOWN_CAT * 256
S_ROWS2 = N_STATIC * 256
KNOWN_ROWS = T * KNOWN_F * 256
OBS_ROWS = T * N_OBS * 256

_mesh = plsc.VectorSubcoreMesh(core_axis_name="c", subcore_axis_name="s")


def _fire(tab, idx4, rows_v, sem):
    for j in range(4):
        pltpu.async_copy(
            tab.at[idx4.at[j]], rows_v.at[pl.ds(j * 128, 128)], sem
        )


def _drain_gathers(tab, idx4, rows_v, sem):
    for j in range(4):
        pltpu.make_async_copy(
            tab.at[idx4.at[j]], rows_v.at[pl.ds(j * 128, 128)], sem
        ).wait()


def _transpose(rows_v, tb):
    lane = lax.iota(jnp.int32, 16)
    i0a = lane // 8
    i1a = lane % 8
    zero = jnp.zeros((16,), jnp.int32)

    def bgl_body(bgl, _2):
        d1b = i1a + bgl * 8

        def blo_body(blo, __):
            r = bgl * 128 + blo
            v0 = rows_v[r, pl.ds(0, 16)]
            v1 = rows_v[r, pl.ds(16, 16)]
            d2 = zero + blo
            plsc.store_scatter(tb, [i0a, d1b, d2], v0)
            plsc.store_scatter(tb, [i0a + 2, d1b, d2], v1)
            return __

        return lax.fori_loop(0, 128, blo_body, _2, unroll=4)

    lax.fori_loop(0, 4, bgl_body, 0)


def _writeback(tb, out_ref, base, sem):
    for lg in range(4):
        pltpu.async_copy(
            tb.at[lg], out_ref.at[pl.ds(base + lg * 64, 32)], sem
        )


def _drain_writes(tb, out_ref, sem):
    for lg in range(4):
        pltpu.make_async_copy(
            tb.at[lg], out_ref.at[pl.ds(lg * 64, 32)], sem
        ).wait()


@functools.partial(
    pl.kernel,
    mesh=_mesh,
    out_type=[
        jax.ShapeDtypeStruct((G_ROWS, 128), jnp.float32),
        jax.ShapeDtypeStruct((S_ROWS2, 128), jnp.float32),
    ],
    scratch_types=[
        pltpu.VMEM((2, 8, 2, 128), jnp.int32),
        pltpu.VMEM((2, 4, 128), jnp.int32),
        pltpu.VMEM((2, 512, 32), jnp.float32),
        pltpu.VMEM((2, 4, 32, 128), jnp.float32),
        pltpu.VMEM((8, 4, 128), jnp.int32),
        pltpu.SemaphoreType.DMA,
        pltpu.SemaphoreType.DMA,
    ],
    compiler_params=pltpu.CompilerParams(
        use_tc_tiling_on_sc=False, needs_layout_passes=False),
)
def _sc_gather(ktab, kidx, stab, sidx, g_out, s_out,
               idx_t, idx4, rows_v, tb, sidx_v, sem_g, sem_o):
    wid = lax.axis_index("s") * NC + lax.axis_index("c")

    def stage_a(u, p):
        t = u // 4
        ct = (u % 4) // 2
        bh = u % 2
        pltpu.sync_copy(kidx.at[t], idx_t.at[p])
        for j in range(4):
            for ch in range(8):
                idx4[p, j, pl.ds(ch * 16, 16)] = (
                    idx_t[p, bh * 4 + j, ct, pl.ds(ch * 16, 16)] + ct * V
                )
        _fire(ktab, idx4.at[p], rows_v.at[p], sem_g)

    def stage_b(u, p, k):
        _drain_gathers(ktab, idx4.at[p], rows_v.at[p], sem_g)

        @pl.when(k >= 2)
        def _():
            _drain_writes(tb.at[p], g_out, sem_o)

        _transpose(rows_v.at[p], tb.at[p])
        t2b = u // 2
        bh = u % 2
        _writeback(tb.at[p], g_out, t2b * 256 + bh * 32, sem_o)

    u0 = wid * HU_PER_W
    stage_a(u0, 0)

    def unit_body(k, c):
        p = k % 2

        @pl.when(k + 1 < HU_PER_W)
        def _():
            stage_a(u0 + k + 1, (k + 1) % 2)

        stage_b(u0 + k, p, k)
        return c

    lax.fori_loop(0, HU_PER_W, unit_body, 0)
    for p in range(2):
        _drain_writes(tb.at[p], g_out, sem_o)

    @pl.when(wid < 8)
    def _():
        f = wid // 2
        bh = wid % 2
        pltpu.sync_copy(sidx, sidx_v)
        for j in range(4):
            for ch in range(8):
                idx4[0, j, pl.ds(ch * 16, 16)] = (
                    sidx_v[bh * 4 + j, f, pl.ds(ch * 16, 16)] + f * V
                )
        _fire(stab, idx4.at[0], rows_v.at[0], sem_g)
        _drain_gathers(stab, idx4.at[0], rows_v.at[0], sem_g)
        _transpose(rows_v.at[0], tb.at[0])
        _writeback(tb.at[0], s_out, f * 256 + bh * 32, sem_o)
        _drain_writes(tb.at[0], s_out, sem_o)


def _known_body(kr_ref, g_ref, w_ref, b_ref, o_ref):
    kr = kr_ref[...]
    krt = kr.reshape(8, 4, 128).transpose(1, 0, 2)
    kr_exp = jnp.broadcast_to(
        krt.reshape(4, 1, 8, 1, 128), (4, 4, 8, 8, 128)
    ).reshape(1024, 128)
    o_ref[pl.ds(0, 1024), :] = w_ref[...] * kr_exp + b_ref[...]
    o_ref[pl.ds(1024, 512), :] = g_ref[...]


def _obs_body(x_ref, w_ref, b_ref, o_ref):
    x = x_ref[...]
    xt = x.reshape(3, 8, 8, 128).transpose(2, 0, 1, 3)
    x_exp = jnp.broadcast_to(
        xt.reshape(8, 3, 1, 8, 1, 128), (8, 3, 4, 8, 8, 128)
    ).reshape(6144, 128)
    o_ref[...] = w_ref[...] * x_exp + b_ref[...]


def kernel(static, known_real, known_categorical, observed, static_tables,
           known_tables, known_dense_w, known_dense_b, observed_dense_w,
           observed_dense_b):
    f32, i32 = jnp.float32, jnp.int32

    kidxN = (known_categorical.astype(i32)
             .reshape(8, 128, T, N_KNOWN_CAT).transpose(2, 0, 3, 1))
    sidxN = (static[:, 0, :].astype(i32)
             .reshape(8, 128, N_STATIC).transpose(0, 2, 1))
    krN = (known_real.reshape(8, 128, T, N_KNOWN_REAL)
           .transpose(2, 0, 3, 1).reshape(T * 32, 128))
    obsN = (observed.reshape(8, 128, 25, 8, N_OBS)
            .transpose(4, 2, 0, 3, 1))

    ktab = known_tables.reshape(N_KNOWN_CAT * V, L)
    stab = static_tables.reshape(N_STATIC * V, L)

    w = known_dense_w.reshape(N_KNOWN_REAL, L)
    bw = known_dense_b.reshape(N_KNOWN_REAL, L)
    w_big = jnp.broadcast_to(
        w.reshape(4, 4, 1, 8, 1), (4, 4, 8, 8, 128)).reshape(1024, 128)
    b_big = jnp.broadcast_to(
        bw.reshape(4, 4, 1, 8, 1), (4, 4, 8, 8, 128)).reshape(1024, 128)
    wo = observed_dense_w.reshape(N_OBS, L)
    bo = observed_dense_b.reshape(N_OBS, L)
    wo_big = jnp.broadcast_to(
        wo.reshape(1, 3, 4, 1, 8, 1), (8, 3, 4, 8, 8, 128)).reshape(6144, 128)
    bo_big = jnp.broadcast_to(
        bo.reshape(1, 3, 4, 1, 8, 1), (8, 3, 4, 8, 8, 128)).reshape(6144, 128)

    g2, s2 = _sc_gather(ktab, kidxN, stab, sidxN)

    out_o = pl.pallas_call(
        _obs_body,
        grid=(25,),
        in_specs=[
            pl.BlockSpec((3, 1, 8, 8, 128), lambda i: (0, i, 0, 0, 0)),
            pl.BlockSpec((6144, 128), lambda i: (0, 0)),
            pl.BlockSpec((6144, 128), lambda i: (0, 0)),
        ],
        out_specs=pl.BlockSpec((6144, 128), lambda i: (i, 0)),
        out_shape=jax.ShapeDtypeStruct((OBS_ROWS, 128), f32),
    )(obsN, wo_big, bo_big)

    out2 = pl.pallas_call(
        _known_body,
        grid=(T,),
        in_specs=[
            pl.BlockSpec((32, 128), lambda i: (i, 0)),
            pl.BlockSpec((512, 128), lambda i: (i, 0)),
            pl.BlockSpec((1024, 128), lambda i: (0, 0)),
            pl.BlockSpec((1024, 128), lambda i: (0, 0)),
        ],
        out_specs=pl.BlockSpec((1536, 128), lambda i: (i, 0)),
        out_shape=jax.ShapeDtypeStruct((KNOWN_ROWS, 128), f32),
    )(krN, g2, w_big, b_big)

    known = (out2.reshape(T, KNOWN_F, 4, 8, 8, 128)
             .transpose(3, 5, 0, 2, 4, 1).reshape(B, T, L, KNOWN_F))
    observed_embeds = (out_o.reshape(T, N_OBS, 4, 8, 8, 128)
                       .transpose(3, 5, 0, 2, 4, 1).reshape(B, T, L, N_OBS))
    static_embeds = (s2.reshape(N_STATIC, 4, 8, 8, 128)
                     .transpose(2, 4, 0, 1, 3).reshape(B, N_STATIC, L))
    return (static_embeds, known, observed_embeds)


# speedup vs baseline: 3.8845x; 1.2578x over previous
import functools

import jax
import jax.numpy as jnp
from jax import lax
from jax.experimental import pallas as pl
from jax.experimental.pallas import tpu as pltpu
from jax.experimental.pallas import tpu_sc as plsc

B, T, L, V = 1024, 200, 32, 100000
BT = B * T
N_STATIC, N_KNOWN_CAT, N_KNOWN_REAL, N_OBS = 4, 2, 4, 3
KNOWN_F = N_KNOWN_REAL + N_KNOWN_CAT

NC, NS = 2, 16
NW = NC * NS

HU_PER_W = (T * N_KNOWN_CAT * 2) // NW

G_ROWS = T * N_KNOWN_CAT * 256
S_ROWS2 = N_STATIC * 256
KNOWN_ROWS = T * KNOWN_F * 256
OBS_ROWS = T * N_OBS * 256

_mesh = plsc.VectorSubcoreMesh(core_axis_name="c", subcore_axis_name="s")


def _transpose(rows_v, tb):
    lane = lax.iota(jnp.int32, 16)
    i0a = lane // 8
    i1a = lane % 8
    zero = jnp.zeros((16,), jnp.int32)

    def bgl_body(bgl, _2):
        d1b = i1a + bgl * 8

        def blo_body(blo, __):
            r = bgl * 128 + blo
            v0 = rows_v[r, pl.ds(0, 16)]
            v1 = rows_v[r, pl.ds(16, 16)]
            d2 = zero + blo
            plsc.store_scatter(tb, [i0a, d1b, d2], v0)
            plsc.store_scatter(tb, [i0a + 2, d1b, d2], v1)
            return __

        return lax.fori_loop(0, 128, blo_body, _2, unroll=4)

    lax.fori_loop(0, 4, bgl_body, 0)


def _writeback(tb, out_ref, base, sem):
    for lg in range(4):
        pltpu.async_copy(
            tb.at[lg], out_ref.at[pl.ds(base + lg * 64, 32)], sem
        )


def _drain_writes(tb, out_ref, sem):
    for lg in range(4):
        pltpu.make_async_copy(
            tb.at[lg], out_ref.at[pl.ds(lg * 64, 32)], sem
        ).wait()


@functools.partial(
    pl.kernel,
    mesh=_mesh,
    out_type=[jax.ShapeDtypeStruct((G_ROWS, 128), jnp.float32)],
    scratch_types=[
        pltpu.VMEM((2, 8, 2, 128), jnp.int32),
        pltpu.VMEM((2, 512, 32), jnp.float32),
        pltpu.VMEM((2, 4, 32, 128), jnp.float32),
        pltpu.SemaphoreType.DMA,
        pltpu.SemaphoreType.DMA,
    ],
    compiler_params=pltpu.CompilerParams(
        use_tc_tiling_on_sc=False, needs_layout_passes=False),
)
def _sc_gather(ktab, kidx, g_out, idx_t, rows_v, tb, sem_g, sem_o):
    wid = lax.axis_index("s") * NC + lax.axis_index("c")

    def run_cat(tab, w16, ct):
        def stage_a(u, p):
            t = u // 2
            pltpu.sync_copy(kidx.at[t], idx_t.at[p])
            bh = u % 2
            for j in range(4):
                pltpu.async_copy(
                    tab.at[idx_t.at[p, bh * 4 + j, ct]],
                    rows_v.at[p, pl.ds(j * 128, 128)], sem_g,
                )

        def stage_b(u, p, k):
            bh = u % 2
            for j in range(4):
                pltpu.make_async_copy(
                    tab.at[idx_t.at[p, bh * 4 + j, ct]],
                    rows_v.at[p, pl.ds(j * 128, 128)], sem_g,
                ).wait()

            @pl.when(k >= 2)
            def _():
                _drain_writes(tb.at[p], g_out, sem_o)

            _transpose(rows_v.at[p], tb.at[p])
            t = u // 2
            _writeback(tb.at[p], g_out, (t * 2 + ct) * 256 + bh * 32, sem_o)

        u0 = w16 * HU_PER_W
        stage_a(u0, 0)

        def unit_body(k, c):
            p = k % 2

            @pl.when(k + 1 < HU_PER_W)
            def _():
                stage_a(u0 + k + 1, (k + 1) % 2)

            stage_b(u0 + k, p, k)
            return c

        lax.fori_loop(0, HU_PER_W, unit_body, 0)
        for p in range(2):
            _drain_writes(tb.at[p], g_out, sem_o)

    @pl.when(wid < 16)
    def _():
        run_cat(ktab.at[0], wid, 0)

    @pl.when(wid >= 16)
    def _():
        run_cat(ktab.at[1], wid - 16, 1)


@functools.partial(
    pl.kernel,
    mesh=_mesh,
    out_type=[jax.ShapeDtypeStruct((S_ROWS2, 128), jnp.float32)],
    scratch_types=[
        pltpu.VMEM((512, 32), jnp.float32),
        pltpu.VMEM((4, 32, 128), jnp.float32),
        pltpu.VMEM((8, 4, 128), jnp.int32),
        pltpu.SemaphoreType.DMA,
        pltpu.SemaphoreType.DMA,
    ],
    compiler_params=pltpu.CompilerParams(
        use_tc_tiling_on_sc=False, needs_layout_passes=False),
)
def _sc_static(stab, sidx, s_out, rows_v, tb, sidx_v, sem_g, sem_o):
    wid = lax.axis_index("s") * NC + lax.axis_index("c")
    for f_ in range(N_STATIC):
        @pl.when(wid // 2 == f_)
        def _(f_=f_):
            bh = wid % 2
            pltpu.sync_copy(sidx, sidx_v)
            tab = stab.at[f_]
            for j in range(4):
                pltpu.async_copy(
                    tab.at[sidx_v.at[bh * 4 + j, f_]],
                    rows_v.at[pl.ds(j * 128, 128)], sem_g,
                )
            for j in range(4):
                pltpu.make_async_copy(
                    tab.at[sidx_v.at[bh * 4 + j, f_]],
                    rows_v.at[pl.ds(j * 128, 128)], sem_g,
                ).wait()
            _transpose(rows_v, tb)
            _writeback(tb, s_out, f_ * 256 + bh * 32, sem_o)
            _drain_writes(tb, s_out, sem_o)


def _known_body(kr_ref, g_ref, w_ref, b_ref, o_ref):
    kr = kr_ref[...]
    krt = kr.reshape(8, 4, 128).transpose(1, 0, 2)
    kr_exp = jnp.broadcast_to(
        krt.reshape(4, 1, 8, 1, 128), (4, 4, 8, 8, 128)
    ).reshape(1024, 128)
    o_ref[pl.ds(0, 1024), :] = w_ref[...] * kr_exp + b_ref[...]
    o_ref[pl.ds(1024, 512), :] = g_ref[...]


def _obs_body(x_ref, w_ref, b_ref, o_ref):
    x = x_ref[...]
    xt = x.reshape(3, 8, 8, 128).transpose(2, 0, 1, 3)
    x_exp = jnp.broadcast_to(
        xt.reshape(8, 3, 1, 8, 1, 128), (8, 3, 4, 8, 8, 128)
    ).reshape(6144, 128)
    o_ref[...] = w_ref[...] * x_exp + b_ref[...]


def kernel(static, known_real, known_categorical, observed, static_tables,
           known_tables, known_dense_w, known_dense_b, observed_dense_w,
           observed_dense_b):
    f32, i32 = jnp.float32, jnp.int32

    kidxN = (known_categorical.astype(i32)
             .reshape(8, 128, T, N_KNOWN_CAT).transpose(2, 0, 3, 1))
    sidxN = (static[:, 0, :].astype(i32)
             .reshape(8, 128, N_STATIC).transpose(0, 2, 1))
    krN = (known_real.reshape(8, 128, T, N_KNOWN_REAL)
           .transpose(2, 0, 3, 1).reshape(T * 32, 128))
    obsN = (observed.reshape(8, 128, 25, 8, N_OBS)
            .transpose(4, 2, 0, 3, 1))

    ktab = known_tables
    stab = static_tables

    w = known_dense_w.reshape(N_KNOWN_REAL, L)
    bw = known_dense_b.reshape(N_KNOWN_REAL, L)
    w_big = jnp.broadcast_to(
        w.reshape(4, 4, 1, 8, 1), (4, 4, 8, 8, 128)).reshape(1024, 128)
    b_big = jnp.broadcast_to(
        bw.reshape(4, 4, 1, 8, 1), (4, 4, 8, 8, 128)).reshape(1024, 128)
    wo = observed_dense_w.reshape(N_OBS, L)
    bo = observed_dense_b.reshape(N_OBS, L)
    wo_big = jnp.broadcast_to(
        wo.reshape(1, 3, 4, 1, 8, 1), (8, 3, 4, 8, 8, 128)).reshape(6144, 128)
    bo_big = jnp.broadcast_to(
        bo.reshape(1, 3, 4, 1, 8, 1), (8, 3, 4, 8, 8, 128)).reshape(6144, 128)

    (g2,) = _sc_gather(ktab, kidxN)
    (s2,) = _sc_static(stab, sidxN)

    out_o = pl.pallas_call(
        _obs_body,
        grid=(25,),
        in_specs=[
            pl.BlockSpec((3, 1, 8, 8, 128), lambda i: (0, i, 0, 0, 0)),
            pl.BlockSpec((6144, 128), lambda i: (0, 0)),
            pl.BlockSpec((6144, 128), lambda i: (0, 0)),
        ],
        out_specs=pl.BlockSpec((6144, 128), lambda i: (i, 0)),
        out_shape=jax.ShapeDtypeStruct((OBS_ROWS, 128), f32),
    )(obsN, wo_big, bo_big)

    out2 = pl.pallas_call(
        _known_body,
        grid=(T,),
        in_specs=[
            pl.BlockSpec((32, 128), lambda i: (i, 0)),
            pl.BlockSpec((512, 128), lambda i: (i, 0)),
            pl.BlockSpec((1024, 128), lambda i: (0, 0)),
            pl.BlockSpec((1024, 128), lambda i: (0, 0)),
        ],
        out_specs=pl.BlockSpec((1536, 128), lambda i: (i, 0)),
        out_shape=jax.ShapeDtypeStruct((KNOWN_ROWS, 128), f32),
    )(krN, g2, w_big, b_big)

    known = (out2.reshape(T, KNOWN_F, 4, 8, 8, 128)
             .transpose(3, 5, 0, 2, 4, 1).reshape(B, T, L, KNOWN_F))
    observed_embeds = (out_o.reshape(T, N_OBS, 4, 8, 8, 128)
                       .transpose(3, 5, 0, 2, 4, 1).reshape(B, T, L, N_OBS))
    static_embeds = (s2.reshape(N_STATIC, 4, 8, 8, 128)
                     .transpose(2, 4, 0, 1, 3).reshape(B, N_STATIC, L))
    return (static_embeds, known, observed_embeds)


# speedup vs baseline: 3.8936x; 1.0023x over previous
import functools

import jax
import jax.numpy as jnp
from jax import lax
from jax.experimental import pallas as pl
from jax.experimental.pallas import tpu as pltpu
from jax.experimental.pallas import tpu_sc as plsc

B, T, L, V = 1024, 200, 32, 100000
BT = B * T
N_STATIC, N_KNOWN_CAT, N_KNOWN_REAL, N_OBS = 4, 2, 4, 3
KNOWN_F = N_KNOWN_REAL + N_KNOWN_CAT

NC, NS = 2, 16
NW = NC * NS

HU_PER_W = (T * N_KNOWN_CAT * 2) // NW

G_ROWS = T * N_KNOWN_CAT * 256
S_ROWS2 = N_STATIC * 256
KNOWN_ROWS = T * KNOWN_F * 256
OBS_ROWS = T * N_OBS * 256

_mesh = plsc.VectorSubcoreMesh(core_axis_name="c", subcore_axis_name="s")


def _transpose(rows_v, tb):
    lane = lax.iota(jnp.int32, 16)
    i0a = lane // 8
    i1a = lane % 8
    zero = jnp.zeros((16,), jnp.int32)

    def bgl_body(bgl, _2):
        d1b = i1a + bgl * 8

        def blo_body(blo, __):
            r = bgl * 128 + blo
            v0 = rows_v[r, pl.ds(0, 16)]
            v1 = rows_v[r, pl.ds(16, 16)]
            d2 = zero + blo
            plsc.store_scatter(tb, [i0a, d1b, d2], v0)
            plsc.store_scatter(tb, [i0a + 2, d1b, d2], v1)
            return __

        return lax.fori_loop(0, 128, blo_body, _2, unroll=4)

    lax.fori_loop(0, 4, bgl_body, 0)


def _writeback(tb, out_ref, base, sem):
    for lg in range(4):
        pltpu.async_copy(
            tb.at[lg], out_ref.at[pl.ds(base + lg * 64, 32)], sem
        )


def _drain_writes(tb, out_ref, sem):
    for lg in range(4):
        pltpu.make_async_copy(
            tb.at[lg], out_ref.at[pl.ds(lg * 64, 32)], sem
        ).wait()


@functools.partial(
    pl.kernel,
    mesh=_mesh,
    out_type=[jax.ShapeDtypeStruct((G_ROWS, 128), jnp.float32)],
    scratch_types=[
        pltpu.VMEM((3, 8, 2, 128), jnp.int32),
        pltpu.VMEM((3, 512, 32), jnp.float32),
        pltpu.VMEM((2, 4, 32, 128), jnp.float32),
        pltpu.SemaphoreType.DMA,
        pltpu.SemaphoreType.DMA,
    ],
    compiler_params=pltpu.CompilerParams(
        use_tc_tiling_on_sc=False, needs_layout_passes=False),
)
def _sc_gather(ktab, kidx, g_out, idx_t, rows_v, tb, sem_g, sem_o):
    wid = lax.axis_index("s") * NC + lax.axis_index("c")

    def run_cat(tab, w16, ct):
        def stage_a(u, p):
            t = u // 2
            pltpu.sync_copy(kidx.at[t], idx_t.at[p])
            bh = u % 2
            for j in range(4):
                pltpu.async_copy(
                    tab.at[idx_t.at[p, bh * 4 + j, ct]],
                    rows_v.at[p, pl.ds(j * 128, 128)], sem_g,
                )

        def stage_b(u, p, pt, k):
            bh = u % 2
            for j in range(4):
                pltpu.make_async_copy(
                    tab.at[idx_t.at[p, bh * 4 + j, ct]],
                    rows_v.at[p, pl.ds(j * 128, 128)], sem_g,
                ).wait()

            @pl.when(k >= 2)
            def _():
                _drain_writes(tb.at[pt], g_out, sem_o)

            _transpose(rows_v.at[p], tb.at[pt])
            t = u // 2
            _writeback(tb.at[pt], g_out, (t * 2 + ct) * 256 + bh * 32, sem_o)

        u0 = w16 * HU_PER_W
        stage_a(u0, 0)
        stage_a(u0 + 1, 1)

        def unit_body(k, c):
            @pl.when(k + 2 < HU_PER_W)
            def _():
                stage_a(u0 + k + 2, (k + 2) % 3)

            stage_b(u0 + k, k % 3, k % 2, k)
            return c

        lax.fori_loop(0, HU_PER_W, unit_body, 0)
        for pt in range(2):
            _drain_writes(tb.at[pt], g_out, sem_o)

    @pl.when(wid < 16)
    def _():
        run_cat(ktab.at[0], wid, 0)

    @pl.when(wid >= 16)
    def _():
        run_cat(ktab.at[1], wid - 16, 1)


@functools.partial(
    pl.kernel,
    mesh=_mesh,
    out_type=[jax.ShapeDtypeStruct((S_ROWS2, 128), jnp.float32)],
    scratch_types=[
        pltpu.VMEM((512, 32), jnp.float32),
        pltpu.VMEM((4, 32, 128), jnp.float32),
        pltpu.VMEM((8, 4, 128), jnp.int32),
        pltpu.SemaphoreType.DMA,
        pltpu.SemaphoreType.DMA,
    ],
    compiler_params=pltpu.CompilerParams(
        use_tc_tiling_on_sc=False, needs_layout_passes=False),
)
def _sc_static(stab, sidx, s_out, rows_v, tb, sidx_v, sem_g, sem_o):
    wid = lax.axis_index("s") * NC + lax.axis_index("c")
    for f_ in range(N_STATIC):
        @pl.when(wid // 2 == f_)
        def _(f_=f_):
            bh = wid % 2
            pltpu.sync_copy(sidx, sidx_v)
            tab = stab.at[f_]
            for j in range(4):
                pltpu.async_copy(
                    tab.at[sidx_v.at[bh * 4 + j, f_]],
                    rows_v.at[pl.ds(j * 128, 128)], sem_g,
                )
            for j in range(4):
                pltpu.make_async_copy(
                    tab.at[sidx_v.at[bh * 4 + j, f_]],
                    rows_v.at[pl.ds(j * 128, 128)], sem_g,
                ).wait()
            _transpose(rows_v, tb)
            _writeback(tb, s_out, f_ * 256 + bh * 32, sem_o)
            _drain_writes(tb, s_out, sem_o)


def _known_body(kr_ref, g_ref, w_ref, b_ref, o_ref):
    kr = kr_ref[...]
    krt = kr.reshape(8, 4, 128).transpose(1, 0, 2)
    kr_exp = jnp.broadcast_to(
        krt.reshape(4, 1, 8, 1, 128), (4, 4, 8, 8, 128)
    ).reshape(1024, 128)
    o_ref[pl.ds(0, 1024), :] = w_ref[...] * kr_exp + b_ref[...]
    o_ref[pl.ds(1024, 512), :] = g_ref[...]


def _obs_body(x_ref, w_ref, b_ref, o_ref):
    x = x_ref[...]
    xt = x.reshape(3, 8, 8, 128).transpose(2, 0, 1, 3)
    x_exp = jnp.broadcast_to(
        xt.reshape(8, 3, 1, 8, 1, 128), (8, 3, 4, 8, 8, 128)
    ).reshape(6144, 128)
    o_ref[...] = w_ref[...] * x_exp + b_ref[...]


def kernel(static, known_real, known_categorical, observed, static_tables,
           known_tables, known_dense_w, known_dense_b, observed_dense_w,
           observed_dense_b):
    f32, i32 = jnp.float32, jnp.int32

    kidxN = (known_categorical.astype(i32)
             .reshape(8, 128, T, N_KNOWN_CAT).transpose(2, 0, 3, 1))
    sidxN = (static[:, 0, :].astype(i32)
             .reshape(8, 128, N_STATIC).transpose(0, 2, 1))
    krN = (known_real.reshape(8, 128, T, N_KNOWN_REAL)
           .transpose(2, 0, 3, 1).reshape(T * 32, 128))
    obsN = (observed.reshape(8, 128, 25, 8, N_OBS)
            .transpose(4, 2, 0, 3, 1))

    ktab = known_tables
    stab = static_tables

    w = known_dense_w.reshape(N_KNOWN_REAL, L)
    bw = known_dense_b.reshape(N_KNOWN_REAL, L)
    w_big = jnp.broadcast_to(
        w.reshape(4, 4, 1, 8, 1), (4, 4, 8, 8, 128)).reshape(1024, 128)
    b_big = jnp.broadcast_to(
        bw.reshape(4, 4, 1, 8, 1), (4, 4, 8, 8, 128)).reshape(1024, 128)
    wo = observed_dense_w.reshape(N_OBS, L)
    bo = observed_dense_b.reshape(N_OBS, L)
    wo_big = jnp.broadcast_to(
        wo.reshape(1, 3, 4, 1, 8, 1), (8, 3, 4, 8, 8, 128)).reshape(6144, 128)
    bo_big = jnp.broadcast_to(
        bo.reshape(1, 3, 4, 1, 8, 1), (8, 3, 4, 8, 8, 128)).reshape(6144, 128)

    (g2,) = _sc_gather(ktab, kidxN)
    (s2,) = _sc_static(stab, sidxN)

    out_o = pl.pallas_call(
        _obs_body,
        grid=(25,),
        in_specs=[
            pl.BlockSpec((3, 1, 8, 8, 128), lambda i: (0, i, 0, 0, 0)),
            pl.BlockSpec((6144, 128), lambda i: (0, 0)),
            pl.BlockSpec((6144, 128), lambda i: (0, 0)),
        ],
        out_specs=pl.BlockSpec((6144, 128), lambda i: (i, 0)),
        out_shape=jax.ShapeDtypeStruct((OBS_ROWS, 128), f32),
    )(obsN, wo_big, bo_big)

    out2 = pl.pallas_call(
        _known_body,
        grid=(T,),
        in_specs=[
            pl.BlockSpec((32, 128), lambda i: (i, 0)),
            pl.BlockSpec((512, 128), lambda i: (i, 0)),
            pl.BlockSpec((1024, 128), lambda i: (0, 0)),
            pl.BlockSpec((1024, 128), lambda i: (0, 0)),
        ],
        out_specs=pl.BlockSpec((1536, 128), lambda i: (i, 0)),
        out_shape=jax.ShapeDtypeStruct((KNOWN_ROWS, 128), f32),
    )(krN, g2, w_big, b_big)

    known = (out2.reshape(T, KNOWN_F, 4, 8, 8, 128)
             .transpose(3, 5, 0, 2, 4, 1).reshape(B, T, L, KNOWN_F))
    observed_embeds = (out_o.reshape(T, N_OBS, 4, 8, 8, 128)
                       .transpose(3, 5, 0, 2, 4, 1).reshape(B, T, L, N_OBS))
    static_embeds = (s2.reshape(N_STATIC, 4, 8, 8, 128)
                     .transpose(2, 4, 0, 1, 3).reshape(B, N_STATIC, L))
    return (static_embeds, known, observed_embeds)
